# baseline (speedup 1.0000x reference)
import functools
import math

import jax
import jax.numpy as jnp
from jax import lax
from jax.experimental import pallas as pl
from jax.experimental.pallas import tpu as pltpu
from jax.experimental.pallas import tpu_sc as plsc

N = 10000
E = 320000
NE = N + E
EP = 330240
EPT = EP // 32
NBUCK = 320
NBUCKP = 336
BW = 32
CAP = 1344
BN = 1000

DNB = 80
DEPT = DNB * 128
DEP = 32 * DEPT
NPADD = 10112
RPTD = NPADD // 16


def _sc_mesh():
    return plsc.VectorSubcoreMesh(core_axis_name="c", subcore_axis_name="s")


_SC_PARAMS = pltpu.CompilerParams(use_tc_tiling_on_sc=False)
if "needs_layout_passes" in pltpu.CompilerParams.__dataclass_fields__:
    import dataclasses as _dc
    _SC_PARAMS = _dc.replace(_SC_PARAMS, needs_layout_passes=False)


def _make_deg():
    @functools.partial(
        pl.kernel,
        mesh=_sc_mesh(),
        compiler_params=_SC_PARAMS,
        out_type=jax.ShapeDtypeStruct((2, NPADD, 16), jnp.float32),
        scratch_types=[
            pltpu.VMEM((DNB, 128), jnp.int32),
            pltpu.VMEM((128, 16), jnp.float32),
            pltpu.VMEM((RPTD, 16), jnp.float32),
            pltpu.VMEM_SHARED((NPADD, 16), jnp.float32),
        ],
    )
    def deg_kernel(dstp_hbm, ones_hbm, z_hbm, deg_hbm, dst_v, ones_v, z_v, acc):
        c = lax.axis_index("c")
        s = lax.axis_index("s")
        pltpu.sync_copy(dstp_hbm.at[c, s], dst_v)
        pltpu.sync_copy(ones_hbm, ones_v)
        pltpu.sync_copy(z_hbm, z_v)
        row0 = s * RPTD
        pltpu.sync_copy(z_v, acc.at[pl.ds(row0, RPTD)])
        plsc.subcore_barrier()

        @pl.loop(0, DNB)
        def _(b):
            pltpu.sync_copy(ones_v, acc.at[dst_v.at[b]], add=True)

        plsc.subcore_barrier()
        pltpu.sync_copy(acc.at[pl.ds(row0, RPTD)], deg_hbm.at[c].at[pl.ds(row0, RPTD)])

    return deg_kernel


def _make_bhist():
    @functools.partial(
        pl.kernel,
        mesh=_sc_mesh(),
        compiler_params=_SC_PARAMS,
        out_type=jax.ShapeDtypeStruct((32, NBUCKP), jnp.int32),
        scratch_types=[
            pltpu.VMEM((EPT,), jnp.int32),
            pltpu.VMEM((NBUCKP,), jnp.int32),
            pltpu.SMEM((NBUCKP,), jnp.int32),
        ],
    )
    def bhist_kernel(dst_hbm, grid_hbm, dst_v, hist_v, hist_sm):
        c = lax.axis_index("c")
        s = lax.axis_index("s")
        t = c * 16 + s
        pltpu.sync_copy(dst_hbm.at[t], dst_v)
        iota = lax.iota(jnp.int32, 16)

        @pl.loop(0, NBUCKP)
        def _(i):
            hist_sm[i] = 0

        @pl.loop(0, EPT // 16)
        def _(g):
            b = dst_v[pl.ds(g * 16, 16)] >> 5
            for k in range(16):
                bk = b[k]
                hist_sm[bk] = hist_sm[bk] + 1

        @pl.loop(0, NBUCKP // 16)
        def _(i):
            z = jnp.zeros((16,), jnp.int32)
            for k in range(16):
                z = jnp.where(iota == k, hist_sm[i * 16 + k], z)
            hist_v[pl.ds(i * 16, 16)] = z

        pltpu.sync_copy(hist_v, grid_hbm.at[t])

    return bhist_kernel


def _make_bperm():
    @functools.partial(
        pl.kernel,
        mesh=_sc_mesh(),
        compiler_params=_SC_PARAMS,
        out_type=[
            jax.ShapeDtypeStruct((NBUCKP * CAP,), jnp.int32),
            jax.ShapeDtypeStruct((NBUCKP * CAP,), jnp.int32),
            jax.ShapeDtypeStruct((NBUCKP * CAP,), jnp.float32),
            jax.ShapeDtypeStruct((32, 16), jnp.int32),
        ],
        scratch_types=[
            pltpu.VMEM((EPT,), jnp.int32),
            pltpu.VMEM((EPT,), jnp.int32),
            pltpu.VMEM((EPT,), jnp.int32),
            pltpu.VMEM((EPT,), jnp.int32),
            pltpu.VMEM((EPT,), jnp.float32),
            pltpu.VMEM((EPT,), jnp.float32),
            pltpu.VMEM((32, NBUCKP), jnp.int32),
            pltpu.VMEM((16,), jnp.int32),
            pltpu.VMEM((16,), jnp.int32),
            pltpu.VMEM((16,), jnp.int32),
            pltpu.VMEM((16,), jnp.float32),
            pltpu.SMEM((NBUCKP,), jnp.int32),
            pltpu.SemaphoreType.DMA,
        ],
    )
    def bperm_kernel(src_hbm, dst_hbm, grid_hbm, dis_hbm, psrc_hbm, ploc_hbm,
                     pnrm_hbm, srcb_hbm, locb_hbm, normb_hbm, cnt_hbm,
                     src_v, dst_v, loc_v, pos_v, dsrc_v, dnrm_v, grid_v,
                     mycnt_v, psrc_v, ploc_v, pnrm_v, off_sm, sem):
        c = lax.axis_index("c")
        s = lax.axis_index("s")
        t = c * 16 + s
        pltpu.sync_copy(src_hbm.at[t], src_v)
        pltpu.sync_copy(dst_hbm.at[t], dst_v)
        pltpu.sync_copy(grid_hbm, grid_v)
        iota = lax.iota(jnp.int32, 16)

        @pl.loop(0, NBUCKP // 16)
        def _(i):
            sl = pl.ds(i * 16, 16)
            tot = jnp.zeros((16,), jnp.int32)
            for tp in range(32):
                g = grid_v[tp, sl]
                tot = tot + jnp.where(tp < t, g, 0)
            for k in range(16):
                off_sm[i * 16 + k] = tot[k]

        @pl.loop(0, EPT // 16)
        def _(g):
            sl = pl.ds(g * 16, 16)
            d = dst_v[sl]
            b = d >> 5
            loc_v[sl] = d & 31
            basep = b * CAP
            z = jnp.zeros((16,), jnp.int32)
            for k in range(16):
                bk = b[k]
                o = off_sm[bk]
                off_sm[bk] = o + 1
                z = jnp.where(iota == k, basep + o, z)
            pos_v[sl] = z

        pltpu.async_copy(dis_hbm.at[src_v], dsrc_v, sem).wait()
        pltpu.async_copy(dis_hbm.at[dst_v], dnrm_v, sem).wait()

        @pl.loop(0, EPT // 16)
        def _(i):
            sl = pl.ds(i * 16, 16)
            dnrm_v[sl] = dsrc_v[sl] * dnrm_v[sl]

        for v, hb in ((src_v, srcb_hbm), (loc_v, locb_hbm), (dnrm_v, normb_hbm)):
            @pl.loop(0, EPT // 128)
            def _(i, v=v, hb=hb):
                sl = pl.ds(i * 128, 128)
                pltpu.sync_copy(v.at[sl], hb.at[pos_v.at[sl]])
            tl = pl.ds((EPT // 128) * 128, EPT % 128)
            pltpu.sync_copy(v.at[tl], hb.at[pos_v.at[tl]])

        sl = pl.ds(t * 10, 16)
        tot = jnp.zeros((16,), jnp.int32)
        for tp in range(32):
            tot = tot + grid_v[tp, sl]
        mycnt_v[...] = tot
        pltpu.sync_copy(mycnt_v, cnt_hbm.at[t])
        pltpu.sync_copy(psrc_hbm, psrc_v)
        pltpu.sync_copy(ploc_hbm, ploc_v)
        pltpu.sync_copy(pnrm_hbm, pnrm_v)
        for j in range(10):
            b = t * 10 + j
            cj = tot[j]
            for q in range(4):
                idxv = jnp.full((16,), b * CAP + q * 16, jnp.int32) + cj + iota
                pltpu.sync_copy(psrc_v, srcb_hbm.at[idxv])
                pltpu.sync_copy(ploc_v, locb_hbm.at[idxv])
                pltpu.sync_copy(pnrm_v, normb_hbm.at[idxv])

    return bperm_kernel


def _make_prop(F):
    GB = {256: 32, 512: 32, 768: 16, 1024: 16}[F]
    CC = F // 16

    @functools.partial(
        pl.kernel,
        mesh=_sc_mesh(),
        compiler_params=_SC_PARAMS,
        out_type=jax.ShapeDtypeStruct((NBUCK * BW * F,), jnp.float32),
        scratch_types=[
            pltpu.VMEM((CAP,), jnp.int32),
            pltpu.VMEM((CAP,), jnp.int32),
            pltpu.VMEM((CAP,), jnp.float32),
            pltpu.VMEM((512,), jnp.int32),
            pltpu.VMEM((GB, F), jnp.float32),
            pltpu.VMEM(((BW + 1) * F,), jnp.float32),
            pltpu.SemaphoreType.DMA,
        ],
    )
    def prop_kernel(hw_hbm, srcb_hbm, locb_hbm, normb_hbm, cnt_hbm, t_hbm,
                    srcl, locl, norml, cnt_v, gbuf, acc, sem):
        c = lax.axis_index("c")
        s = lax.axis_index("s")
        t = c * 16 + s
        pltpu.sync_copy(cnt_hbm, cnt_v)
        iota = lax.iota(jnp.int32, 16)

        @pl.loop(0, 10)
        def _(j):
            b = t * 10 + j
            pltpu.sync_copy(srcb_hbm.at[pl.ds(b * CAP, CAP)], srcl)
            pltpu.sync_copy(locb_hbm.at[pl.ds(b * CAP, CAP)], locl)
            pltpu.sync_copy(normb_hbm.at[pl.ds(b * CAP, CAP)], norml)
            cnt = cnt_v[pl.ds(t * 16 + j, 16)][0]
            nblk = (cnt + (GB - 1)) // GB

            @pl.loop(0, BW + 1)
            def _(r):
                for cc in range(CC):
                    acc[pl.ds(r * F + cc * 16, 16)] = jnp.zeros((16,), jnp.float32)

            def blk(ib, carry):
                pltpu.async_copy(hw_hbm.at[srcl.at[pl.ds(ib * GB, GB)]],
                                 gbuf, sem).wait()
                for e16 in range(GB // 16):
                    lv = locl[pl.ds(ib * GB + e16 * 16, 16)]
                    nv16 = norml[pl.ds(ib * GB + e16 * 16, 16)]
                    for k in range(16):
                        e = e16 * 16 + k
                        base = lv[k] * F
                        nv = jnp.full((16,), nv16[k], jnp.float32)
                        for cc in range(CC):
                            g = gbuf[e, pl.ds(cc * 16, 16)]
                            idx = jnp.full((16,), base + cc * 16, jnp.int32) + iota
                            plsc.addupdate_scatter(acc, [idx], g * nv)
                return carry

            lax.fori_loop(0, nblk, blk, 0)
            pltpu.sync_copy(acc.at[pl.ds(0, BW * F)],
                            t_hbm.at[pl.ds(b * BW * F, BW * F)])

    return prop_kernel


def _mm_body(x_ref, w_ref, o_ref):
    l16 = x_ref[...].astype(jnp.bfloat16)
    o_ref[...] = jnp.dot(l16, w_ref[...], preferred_element_type=jnp.float32)


def _mm(h, w):
    n, fin = h.shape
    fout = w.shape[1]
    return pl.pallas_call(
        _mm_body,
        grid=(n // BN,),
        in_specs=[
            pl.BlockSpec((BN, fin), lambda i: (i, 0)),
            pl.BlockSpec((fin, fout), lambda i: (0, 0)),
        ],
        out_specs=pl.BlockSpec((BN, fout), lambda i: (i, 0)),
        out_shape=jax.ShapeDtypeStruct((n, fout), jnp.float32),
    )(h, w)


def kernel(x1, edge_index, params):
    src = edge_index[0]
    dst = edge_index[1]
    loop = jnp.arange(N, dtype=jnp.int32)

    dpad = DEP - E
    dstd = jnp.concatenate([dst, jnp.full((dpad,), N, jnp.int32)])
    dstd = dstd.reshape(2, 16, DNB, 128)
    ones16 = jnp.ones((128, 16), jnp.float32)
    z16 = jnp.zeros((RPTD, 16), jnp.float32)
    deg_part = _make_deg()(dstd, ones16, z16)
    deg = deg_part[0, :N, 0] + deg_part[1, :N, 0] + 1.0
    dis = jnp.where(deg > 0, 1.0 / jnp.sqrt(deg), 0.0)

    epad = EP - NE
    src_f = jnp.concatenate([src, loop, jnp.zeros((epad,), jnp.int32)])
    dst_f = jnp.concatenate([dst, loop, jnp.full((epad,), NBUCK * BW,
                                                 jnp.int32)])
    src_f = src_f.reshape(32, EPT)
    dst_f = dst_f.reshape(32, EPT)

    grid = _make_bhist()(dst_f)
    dis_pad = jnp.concatenate([dis, jnp.zeros((NBUCKP * BW - N,),
                                              jnp.float32)])
    psrc = jnp.zeros((16,), jnp.int32)
    ploc = jnp.full((16,), BW, jnp.int32)
    pnrm = jnp.zeros((16,), jnp.float32)
    srcb, locb, normb, cnt = _make_bperm()(src_f, dst_f, grid, dis_pad,
                                           psrc, ploc, pnrm)

    h = x1
    for layer in params["conv"]:
        hw = _mm(h, layer["W"])
        F = hw.shape[1]
        tflat = _make_prop(F)(hw, srcb, locb, normb, cnt.reshape(512))
        t = tflat.reshape(NBUCK * BW, F)[:N]
        h = jax.nn.relu(t + layer["b"])
    pooled = jnp.max(h, axis=0, keepdims=True)
    z = pooled
    dec = params["dec"]
    for i in range(3):
        z = z @ dec[i]["W"] + dec[i]["b"]
        z = dec[i]["gamma"] * z / jnp.sqrt(1.0 + 1e-5) + dec[i]["beta"]
        z = jax.nn.relu(z)
    return z @ dec[3]["W"] + dec[3]["b"]


# speedup vs baseline: 1.0414x; 1.0414x over previous
import functools
import math

import jax
import jax.numpy as jnp
from jax import lax
from jax.experimental import pallas as pl
from jax.experimental.pallas import tpu as pltpu
from jax.experimental.pallas import tpu_sc as plsc

N = 10000
E = 320000
NE = N + E
EP = 330240
EPT = EP // 32
NBUCK = 320
NBUCKP = 336
BW = 32
CAP = 1344
BN = 1000

DNB = 80
DEPT = DNB * 128
DEP = 32 * DEPT
NPADD = 10112
RPTD = NPADD // 16


def _sc_mesh():
    return plsc.VectorSubcoreMesh(core_axis_name="c", subcore_axis_name="s")


_SC_PARAMS = pltpu.CompilerParams(use_tc_tiling_on_sc=False)
if "needs_layout_passes" in pltpu.CompilerParams.__dataclass_fields__:
    import dataclasses as _dc
    _SC_PARAMS = _dc.replace(_SC_PARAMS, needs_layout_passes=False)


def _make_deg():
    @functools.partial(
        pl.kernel,
        mesh=_sc_mesh(),
        compiler_params=_SC_PARAMS,
        out_type=jax.ShapeDtypeStruct((2, NPADD, 16), jnp.float32),
        scratch_types=[
            pltpu.VMEM((DNB, 128), jnp.int32),
            pltpu.VMEM((128, 16), jnp.float32),
            pltpu.VMEM((RPTD, 16), jnp.float32),
            pltpu.VMEM_SHARED((NPADD, 16), jnp.float32),
        ],
    )
    def deg_kernel(dstp_hbm, ones_hbm, z_hbm, deg_hbm, dst_v, ones_v, z_v, acc):
        c = lax.axis_index("c")
        s = lax.axis_index("s")
        pltpu.sync_copy(dstp_hbm.at[c, s], dst_v)
        pltpu.sync_copy(ones_hbm, ones_v)
        pltpu.sync_copy(z_hbm, z_v)
        row0 = s * RPTD
        pltpu.sync_copy(z_v, acc.at[pl.ds(row0, RPTD)])
        plsc.subcore_barrier()

        @pl.loop(0, DNB)
        def _(b):
            pltpu.sync_copy(ones_v, acc.at[dst_v.at[b]], add=True)

        plsc.subcore_barrier()
        pltpu.sync_copy(acc.at[pl.ds(row0, RPTD)], deg_hbm.at[c].at[pl.ds(row0, RPTD)])

    return deg_kernel


def _make_bhist():
    @functools.partial(
        pl.kernel,
        mesh=_sc_mesh(),
        compiler_params=_SC_PARAMS,
        out_type=jax.ShapeDtypeStruct((32, NBUCKP), jnp.int32),
        scratch_types=[
            pltpu.VMEM((EPT,), jnp.int32),
            pltpu.VMEM((NBUCKP,), jnp.int32),
            pltpu.SMEM((NBUCKP,), jnp.int32),
        ],
    )
    def bhist_kernel(dst_hbm, grid_hbm, dst_v, hist_v, hist_sm):
        c = lax.axis_index("c")
        s = lax.axis_index("s")
        t = c * 16 + s
        pltpu.sync_copy(dst_hbm.at[t], dst_v)
        iota = lax.iota(jnp.int32, 16)

        @pl.loop(0, NBUCKP)
        def _(i):
            hist_sm[i] = 0

        @pl.loop(0, EPT // 16)
        def _(g):
            b = dst_v[pl.ds(g * 16, 16)] >> 5
            for k in range(16):
                bk = b[k]
                hist_sm[bk] = hist_sm[bk] + 1

        @pl.loop(0, NBUCKP // 16)
        def _(i):
            z = jnp.zeros((16,), jnp.int32)
            for k in range(16):
                z = jnp.where(iota == k, hist_sm[i * 16 + k], z)
            hist_v[pl.ds(i * 16, 16)] = z

        pltpu.sync_copy(hist_v, grid_hbm.at[t])

    return bhist_kernel


def _make_bperm():
    @functools.partial(
        pl.kernel,
        mesh=_sc_mesh(),
        compiler_params=_SC_PARAMS,
        out_type=[
            jax.ShapeDtypeStruct((NBUCKP * CAP,), jnp.int32),
            jax.ShapeDtypeStruct((NBUCKP * CAP,), jnp.int32),
            jax.ShapeDtypeStruct((NBUCKP * CAP,), jnp.float32),
            jax.ShapeDtypeStruct((32, 16), jnp.int32),
        ],
        scratch_types=[
            pltpu.VMEM((EPT,), jnp.int32),
            pltpu.VMEM((EPT,), jnp.int32),
            pltpu.VMEM((EPT,), jnp.int32),
            pltpu.VMEM((EPT,), jnp.int32),
            pltpu.VMEM((EPT,), jnp.float32),
            pltpu.VMEM((EPT,), jnp.float32),
            pltpu.VMEM((32, NBUCKP), jnp.int32),
            pltpu.VMEM((16,), jnp.int32),
            pltpu.VMEM((16,), jnp.int32),
            pltpu.VMEM((16,), jnp.int32),
            pltpu.VMEM((16,), jnp.float32),
            pltpu.SMEM((NBUCKP,), jnp.int32),
            pltpu.SemaphoreType.DMA,
        ],
    )
    def bperm_kernel(src_hbm, dst_hbm, grid_hbm, dis_hbm, psrc_hbm, ploc_hbm,
                     pnrm_hbm, srcb_hbm, locb_hbm, normb_hbm, cnt_hbm,
                     src_v, dst_v, loc_v, pos_v, dsrc_v, dnrm_v, grid_v,
                     mycnt_v, psrc_v, ploc_v, pnrm_v, off_sm, sem):
        c = lax.axis_index("c")
        s = lax.axis_index("s")
        t = c * 16 + s
        pltpu.sync_copy(src_hbm.at[t], src_v)
        pltpu.sync_copy(dst_hbm.at[t], dst_v)
        pltpu.sync_copy(grid_hbm, grid_v)
        iota = lax.iota(jnp.int32, 16)

        @pl.loop(0, NBUCKP // 16)
        def _(i):
            sl = pl.ds(i * 16, 16)
            tot = jnp.zeros((16,), jnp.int32)
            for tp in range(32):
                g = grid_v[tp, sl]
                tot = tot + jnp.where(tp < t, g, 0)
            for k in range(16):
                off_sm[i * 16 + k] = tot[k]

        @pl.loop(0, EPT // 16)
        def _(g):
            sl = pl.ds(g * 16, 16)
            d = dst_v[sl]
            b = d >> 5
            loc_v[sl] = d & 31
            basep = b * CAP
            z = jnp.zeros((16,), jnp.int32)
            for k in range(16):
                bk = b[k]
                o = off_sm[bk]
                off_sm[bk] = o + 1
                z = jnp.where(iota == k, basep + o, z)
            pos_v[sl] = z

        pltpu.async_copy(dis_hbm.at[src_v], dsrc_v, sem).wait()
        pltpu.async_copy(dis_hbm.at[dst_v], dnrm_v, sem).wait()

        @pl.loop(0, EPT // 16)
        def _(i):
            sl = pl.ds(i * 16, 16)
            dnrm_v[sl] = dsrc_v[sl] * dnrm_v[sl]

        for v, hb in ((src_v, srcb_hbm), (loc_v, locb_hbm), (dnrm_v, normb_hbm)):
            @pl.loop(0, EPT // 128)
            def _(i, v=v, hb=hb):
                sl = pl.ds(i * 128, 128)
                pltpu.sync_copy(v.at[sl], hb.at[pos_v.at[sl]])
            tl = pl.ds((EPT // 128) * 128, EPT % 128)
            pltpu.sync_copy(v.at[tl], hb.at[pos_v.at[tl]])

        sl = pl.ds(t * 10, 16)
        tot = jnp.zeros((16,), jnp.int32)
        for tp in range(32):
            tot = tot + grid_v[tp, sl]
        mycnt_v[...] = tot
        pltpu.sync_copy(mycnt_v, cnt_hbm.at[t])
        pltpu.sync_copy(psrc_hbm, psrc_v)
        pltpu.sync_copy(ploc_hbm, ploc_v)
        pltpu.sync_copy(pnrm_hbm, pnrm_v)
        for j in range(10):
            b = t * 10 + j
            cj = tot[j]
            for q in range(4):
                idxv = jnp.full((16,), b * CAP + q * 16, jnp.int32) + cj + iota
                pltpu.sync_copy(psrc_v, srcb_hbm.at[idxv])
                pltpu.sync_copy(ploc_v, locb_hbm.at[idxv])
                pltpu.sync_copy(pnrm_v, normb_hbm.at[idxv])

    return bperm_kernel


def _make_prop(F):
    GB = 16
    CC = F // 16

    @functools.partial(
        pl.kernel,
        mesh=_sc_mesh(),
        compiler_params=_SC_PARAMS,
        out_type=jax.ShapeDtypeStruct((NBUCK * BW * F,), jnp.float32),
        scratch_types=[
            pltpu.VMEM((CAP,), jnp.int32),
            pltpu.VMEM((CAP,), jnp.int32),
            pltpu.VMEM((CAP,), jnp.float32),
            pltpu.VMEM((512,), jnp.int32),
            pltpu.VMEM((GB, F), jnp.float32),
            pltpu.VMEM(((BW + 1) * F,), jnp.float32),
            pltpu.SemaphoreType.DMA,
        ],
    )
    def prop_kernel(hw_hbm, srcb_hbm, locb_hbm, normb_hbm, cnt_hbm, t_hbm,
                    srcl, locl, norml, cnt_v, gbuf, acc, sem):
        c = lax.axis_index("c")
        s = lax.axis_index("s")
        t = c * 16 + s
        pltpu.sync_copy(cnt_hbm, cnt_v)
        iota = lax.iota(jnp.int32, 16)

        @pl.loop(0, 10)
        def _(j):
            b = t * 10 + j
            pltpu.sync_copy(srcb_hbm.at[pl.ds(b * CAP, CAP)], srcl)
            pltpu.sync_copy(locb_hbm.at[pl.ds(b * CAP, CAP)], locl)
            pltpu.sync_copy(normb_hbm.at[pl.ds(b * CAP, CAP)], norml)
            cnt = cnt_v[pl.ds(t * 16 + j, 16)][0]
            nblk = (cnt + (GB - 1)) // GB

            @pl.loop(0, BW + 1)
            def _(r):
                for cc in range(CC):
                    acc[pl.ds(r * F + cc * 16, 16)] = jnp.zeros((16,), jnp.float32)

            def blk(ib, carry):
                pltpu.async_copy(hw_hbm.at[srcl.at[pl.ds(ib * GB, GB)]],
                                 gbuf, sem).wait()
                for e16 in range(GB // 16):
                    lv = locl[pl.ds(ib * GB + e16 * 16, 16)]
                    nv16 = norml[pl.ds(ib * GB + e16 * 16, 16)]
                    for k in range(16):
                        e = e16 * 16 + k
                        base = lv[k] * F
                        nv = jnp.full((16,), nv16[k], jnp.float32)
                        for cc in range(CC):
                            g = gbuf[e, pl.ds(cc * 16, 16)]
                            idx = jnp.full((16,), base + cc * 16, jnp.int32) + iota
                            plsc.addupdate_scatter(acc, [idx], g * nv)
                return carry

            lax.fori_loop(0, nblk, blk, 0)
            pltpu.sync_copy(acc.at[pl.ds(0, BW * F)],
                            t_hbm.at[pl.ds(b * BW * F, BW * F)])

    return prop_kernel


def _mm_body(x_ref, w_ref, o_ref):
    l16 = x_ref[...].astype(jnp.bfloat16)
    o_ref[...] = jnp.dot(l16, w_ref[...], preferred_element_type=jnp.float32)


def _mm(h, w):
    n, fin = h.shape
    fout = w.shape[1]
    return pl.pallas_call(
        _mm_body,
        grid=(n // BN,),
        in_specs=[
            pl.BlockSpec((BN, fin), lambda i: (i, 0)),
            pl.BlockSpec((fin, fout), lambda i: (0, 0)),
        ],
        out_specs=pl.BlockSpec((BN, fout), lambda i: (i, 0)),
        out_shape=jax.ShapeDtypeStruct((n, fout), jnp.float32),
    )(h, w)


def kernel(x1, edge_index, params):
    src = edge_index[0]
    dst = edge_index[1]
    loop = jnp.arange(N, dtype=jnp.int32)

    dpad = DEP - E
    dstd = jnp.concatenate([dst, jnp.full((dpad,), N, jnp.int32)])
    dstd = dstd.reshape(2, 16, DNB, 128)
    ones16 = jnp.ones((128, 16), jnp.float32)
    z16 = jnp.zeros((RPTD, 16), jnp.float32)
    deg_part = _make_deg()(dstd, ones16, z16)
    deg = deg_part[0, :N, 0] + deg_part[1, :N, 0] + 1.0
    dis = jnp.where(deg > 0, 1.0 / jnp.sqrt(deg), 0.0)

    epad = EP - NE
    src_f = jnp.concatenate([src, loop, jnp.zeros((epad,), jnp.int32)])
    dst_f = jnp.concatenate([dst, loop, jnp.full((epad,), NBUCK * BW,
                                                 jnp.int32)])
    src_f = src_f.reshape(32, EPT)
    dst_f = dst_f.reshape(32, EPT)

    grid = _make_bhist()(dst_f)
    dis_pad = jnp.concatenate([dis, jnp.zeros((NBUCKP * BW - N,),
                                              jnp.float32)])
    psrc = jnp.zeros((16,), jnp.int32)
    ploc = jnp.full((16,), BW, jnp.int32)
    pnrm = jnp.zeros((16,), jnp.float32)
    srcb, locb, normb, cnt = _make_bperm()(src_f, dst_f, grid, dis_pad,
                                           psrc, ploc, pnrm)

    h = x1
    for layer in params["conv"]:
        hw = _mm(h, layer["W"])
        F = hw.shape[1]
        tflat = _make_prop(F)(hw, srcb, locb, normb, cnt.reshape(512))
        t = tflat.reshape(NBUCK * BW, F)[:N]
        h = jax.nn.relu(t + layer["b"])
    pooled = jnp.max(h, axis=0, keepdims=True)
    z = pooled
    dec = params["dec"]
    for i in range(3):
        z = z @ dec[i]["W"] + dec[i]["b"]
        z = dec[i]["gamma"] * z / jnp.sqrt(1.0 + 1e-5) + dec[i]["beta"]
        z = jax.nn.relu(z)
    return z @ dec[3]["W"] + dec[3]["b"]
